# baseline (speedup 1.0000x reference)
import functools

import numpy as np

import jax
import jax.numpy as jnp
from jax import lax
from jax.experimental import pallas as pl
from jax.experimental.pallas import tpu as pltpu
from jax.experimental.pallas import tpu_sc as plsc

_NIN = 256
_NOU = 128
_ET = 4
_N = 10000
_K = 16
_D = _NOU * _ET
_DG = _D // 2
_DGE = _NOU // 2

_PERM = np.concatenate([np.arange(0, _NOU, 2), np.arange(1, _NOU, 2)])
_PERM512 = (np.arange(_ET)[:, None] * _NOU + _PERM[None, :]).reshape(-1)
_INV_PERM = np.argsort(_PERM)

_NB_MM = 2000


def _mm_body(x_ref, wn_ref, we_ref, efa_ref, efb_ref, s_ref):
    xb = x_ref[...]
    dn = (((1,), (0,)), ((), ()))
    nf = lax.dot_general(xb, wn_ref[...], dn, preferred_element_type=jnp.float32)
    ef = lax.dot_general(xb, we_ref[...], dn, preferred_element_type=jnp.float32)
    def rne(v):
        b = lax.bitcast_convert_type(v, jnp.int32)
        return b + 0x7FFF + ((b >> 16) & 1)

    for e in range(_ET):
        lo = (rne(ef[:, e * _NOU:e * _NOU + _DGE]) >> 16) & 0xFFFF
        hi = rne(ef[:, e * _NOU + _DGE:(e + 1) * _NOU]) & jnp.int32(-65536)
        ref = efa_ref if e < 2 else efb_ref
        ref[:, (e % 2) * _DGE:((e % 2) + 1) * _DGE] = lo | hi
    s_ref[...] = nf + ef


def _matmul(x, wn, we):
    return pl.pallas_call(
        _mm_body,
        grid=(_N // _NB_MM,),
        in_specs=[
            pl.BlockSpec((_NB_MM, _NIN), lambda i: (i, 0)),
            pl.BlockSpec((_NIN, _D), lambda i: (0, 0)),
            pl.BlockSpec((_NIN, _D), lambda i: (0, 0)),
        ],
        out_specs=[
            pl.BlockSpec((_NB_MM, _NOU), lambda i: (i, 0)),
            pl.BlockSpec((_NB_MM, _NOU), lambda i: (i, 0)),
            pl.BlockSpec((_NB_MM, _D), lambda i: (i, 0)),
        ],
        out_shape=[
            jax.ShapeDtypeStruct((_N, _NOU), jnp.int32),
            jax.ShapeDtypeStruct((_N, _NOU), jnp.int32),
            jax.ShapeDtypeStruct((_N, _D), jnp.float32),
        ],
    )(x, wn, we)


_NC = 2
_NS = 16
_NW = _NC * _NS
_KH = _K // 2
_CH = 120
_NCHUNK = 21
_NOUTER = _NCHUNK // 3
_BPW = _CH * _NCHUNK
_EPH = _BPW * _NW
_NPAD = _EPH // _KH


def _make_gather():
    mesh = plsc.VectorSubcoreMesh(core_axis_name="c", subcore_axis_name="s")

    @functools.partial(
        pl.kernel,
        mesh=mesh,
        out_type=[
            jax.ShapeDtypeStruct((_EPH, _NOU), jnp.int32),
            jax.ShapeDtypeStruct((_EPH, _NOU), jnp.int32),
        ],
        scratch_types=[
            pltpu.VMEM((_BPW,), jnp.int32),
            pltpu.VMEM((3, _CH, _NOU), jnp.int32),
            pltpu.VMEM((3, _CH, _NOU), jnp.int32),
            pltpu.SemaphoreType.DMA((3,)),
            pltpu.SemaphoreType.DMA((3,)),
        ],
    )
    def gather_k(ta_hbm, tb_hbm, idx_hbm, oa_hbm, ob_hbm, idx_v, ra_v, rb_v, gsem, wsem):
        wid = lax.axis_index("s") * _NC + lax.axis_index("c")
        base = wid * _BPW
        pltpu.sync_copy(idx_hbm.at[pl.ds(base, _BPW)], idx_v)

        def fire_gather(g, slot):
            sl = idx_v.at[pl.ds(g * _CH, _CH)]
            pltpu.async_copy(ta_hbm.at[sl], ra_v.at[slot], gsem.at[slot])
            pltpu.async_copy(tb_hbm.at[sl], rb_v.at[slot], gsem.at[slot])

        def wait_gather(g, slot):
            sl = idx_v.at[pl.ds(g * _CH, _CH)]
            pltpu.make_async_copy(ta_hbm.at[sl], ra_v.at[slot], gsem.at[slot]).wait()
            pltpu.make_async_copy(tb_hbm.at[sl], rb_v.at[slot], gsem.at[slot]).wait()

        def fire_write(g, slot):
            dst = pl.ds(base + g * _CH, _CH)
            pltpu.async_copy(ra_v.at[slot], oa_hbm.at[dst], wsem.at[slot])
            pltpu.async_copy(rb_v.at[slot], ob_hbm.at[dst], wsem.at[slot])

        def wait_write(slot):
            pltpu.make_async_copy(
                ra_v.at[slot], oa_hbm.at[pl.ds(0, _CH)], wsem.at[slot]
            ).wait()
            pltpu.make_async_copy(
                rb_v.at[slot], ob_hbm.at[pl.ds(0, _CH)], wsem.at[slot]
            ).wait()

        fire_gather(0, 0)
        fire_gather(1, 1)

        def outer(g3, carry):
            for b in range(3):
                g = g3 * 3 + b
                wait_gather(g, b)
                fire_write(g, b)
                nxt = (b + 2) % 3
                if b == 0:
                    @pl.when(g3 >= 1)
                    def _():
                        wait_write(nxt)
                    fire_gather(g + 2, nxt)
                else:
                    @pl.when(g3 < _NOUTER - 1)
                    def _():
                        wait_write(nxt)
                        fire_gather(g + 2, nxt)
            return carry

        lax.fori_loop(0, _NOUTER, outer, 0)
        for b in range(3):
            wait_write(b)

    return gather_k


_gather_cache = []


def _gather(table_a, table_b, idx):
    if not _gather_cache:
        _gather_cache.append(_make_gather())
    return _gather_cache[0](table_a, table_b, idx)


_NB_E = 200


def _edge_body(s_ref, pa_ref, pb_ref, et_ref, bias_ref, out_ref):
    s = s_ref[...]
    m_lo = m_hi = None
    for k in range(_KH):
        pka = pa_ref[k]
        pkb = pb_ref[k]
        ek = et_ref[k]
        t_lo = jnp.zeros((_NB_E, _DGE), jnp.float32)
        t_hi = jnp.zeros((_NB_E, _DGE), jnp.float32)
        for e in range(_ET):
            w = ek[:, e:e + 1]
            pk = pka if e < 2 else pkb
            word = pk[:, (e % 2) * _DGE:((e % 2) + 1) * _DGE]
            lo = lax.bitcast_convert_type(word << 16, jnp.float32)
            hi = lax.bitcast_convert_type(word & jnp.int32(-65536), jnp.float32)
            t_lo = t_lo + w * (s[:, e * _NOU:e * _NOU + _DGE] - lo)
            t_hi = t_hi + w * (s[:, e * _NOU + _DGE:(e + 1) * _NOU] - hi)
        if m_lo is None:
            m_lo, m_hi = t_lo, t_hi
        else:
            m_lo = jnp.maximum(m_lo, t_lo)
            m_hi = jnp.maximum(m_hi, t_hi)
    b = bias_ref[...]
    out_ref[:, :_DGE] = m_lo + b[:, :_DGE]
    out_ref[:, _DGE:] = m_hi + b[:, _DGE:]


def _edge(s, pts_a, pts_b, et_km, bias2d):
    return pl.pallas_call(
        _edge_body,
        grid=(_N // _NB_E,),
        in_specs=[
            pl.BlockSpec((_NB_E, _D), lambda i: (i, 0)),
            pl.BlockSpec((_KH, _NB_E, _NOU), lambda i: (0, i, 0)),
            pl.BlockSpec((_KH, _NB_E, _NOU), lambda i: (0, i, 0)),
            pl.BlockSpec((_KH, _NB_E, _ET), lambda i: (0, i, 0)),
            pl.BlockSpec((1, _NOU), lambda i: (0, 0)),
        ],
        out_specs=pl.BlockSpec((_NB_E, _NOU), lambda i: (i, 0)),
        out_shape=jax.ShapeDtypeStruct((_N, _NOU), jnp.float32),
    )(s, pts_a, pts_b, et_km, bias2d)


def _bn_body(o1_ref, o2_ref, g_ref, b_ref, out_ref):
    o = jnp.maximum(o1_ref[...], o2_ref[...])
    mean = jnp.mean(o, axis=0, keepdims=True)
    ctr = o - mean
    var = jnp.mean(ctr * ctr, axis=0, keepdims=True)
    y = ctr * lax.rsqrt(var + 1e-5) * g_ref[...] + b_ref[...]
    out_ref[...] = jnp.maximum(y, 0.0).T


def _bn(o1, o2, g, b):
    return pl.pallas_call(
        _bn_body,
        out_shape=jax.ShapeDtypeStruct((_NOU, _N), jnp.float32),
    )(o1, o2, g, b)


def kernel(x, nn_idx, etype, filters1, filters2, bias, bn_gamma, bn_beta):
    wn = jnp.transpose(filters1, (0, 2, 1)).reshape(_NIN, _D)[:, _PERM512]
    we = jnp.transpose(filters2, (0, 2, 1)).reshape(_NIN, _D)[:, _PERM512]
    ef_a, ef_b, s = _matmul(jnp.transpose(x.reshape(_NIN, _N)), wn, we)

    pad = (jnp.arange(_K * (_NPAD - _N), dtype=jnp.int32) * 7) % _N
    idx_km = jnp.concatenate(
        [
            jnp.transpose(nn_idx.reshape(_N, _K)).astype(jnp.int32),
            pad.reshape(_K, _NPAD - _N),
        ],
        axis=1,
    )
    et_km = jnp.transpose(etype.reshape(_ET, _N, _K), (2, 1, 0))
    bias_p = bias[_PERM].reshape(1, _NOU)

    p1a, p1b = _gather(ef_a, ef_b, idx_km[:_KH].reshape(-1))
    p2a, p2b = _gather(ef_a, ef_b, idx_km[_KH:].reshape(-1))
    sh = (_KH, _NPAD, _NOU)
    o1 = _edge(s, p1a.reshape(sh), p1b.reshape(sh), et_km[:_KH], bias_p)
    o2 = _edge(s, p2a.reshape(sh), p2b.reshape(sh), et_km[_KH:], bias_p)
    out = _bn(
        o1,
        o2,
        bn_gamma[_PERM].reshape(1, _NOU),
        bn_beta[_PERM].reshape(1, _NOU),
    )
    return out[_INV_PERM, :].reshape(1, _NOU, _N, 1)


# speedup vs baseline: 1.3144x; 1.3144x over previous
import functools

import numpy as np

import jax
import jax.numpy as jnp
from jax import lax
from jax.experimental import pallas as pl
from jax.experimental.pallas import tpu as pltpu
from jax.experimental.pallas import tpu_sc as plsc

_NIN = 256
_NOU = 128
_ET = 4
_N = 10000
_K = 16
_D = _NOU * _ET
_DG = _D // 2
_DGE = _NOU // 2

_PERM = np.concatenate([np.arange(0, _NOU, 2), np.arange(1, _NOU, 2)])
_PERM512 = (np.arange(_ET)[:, None] * _NOU + _PERM[None, :]).reshape(-1)
_INV_PERM = np.argsort(_PERM)

_NB_MM = 2000


def _mm_body(x_ref, wn_ref, we_ref, efa_ref, efb_ref, s_ref):
    xb = x_ref[...]
    dn = (((1,), (0,)), ((), ()))
    nf = lax.dot_general(xb, wn_ref[...], dn, preferred_element_type=jnp.float32)
    ef = lax.dot_general(xb, we_ref[...], dn, preferred_element_type=jnp.float32)
    def rne(v):
        b = lax.bitcast_convert_type(v, jnp.int32)
        return b + 0x7FFF + ((b >> 16) & 1)

    for e in range(_ET):
        lo = (rne(ef[:, e * _NOU:e * _NOU + _DGE]) >> 16) & 0xFFFF
        hi = rne(ef[:, e * _NOU + _DGE:(e + 1) * _NOU]) & jnp.int32(-65536)
        ref = efa_ref if e < 2 else efb_ref
        ref[:, (e % 2) * _DGE:((e % 2) + 1) * _DGE] = lo | hi
    s_ref[...] = nf + ef


def _matmul(x, wn, we):
    return pl.pallas_call(
        _mm_body,
        grid=(_N // _NB_MM,),
        in_specs=[
            pl.BlockSpec((_NB_MM, _NIN), lambda i: (i, 0)),
            pl.BlockSpec((_NIN, _D), lambda i: (0, 0)),
            pl.BlockSpec((_NIN, _D), lambda i: (0, 0)),
        ],
        out_specs=[
            pl.BlockSpec((_NB_MM, _NOU), lambda i: (i, 0)),
            pl.BlockSpec((_NB_MM, _NOU), lambda i: (i, 0)),
            pl.BlockSpec((_NB_MM, _D), lambda i: (i, 0)),
        ],
        out_shape=[
            jax.ShapeDtypeStruct((_N, _NOU), jnp.int32),
            jax.ShapeDtypeStruct((_N, _NOU), jnp.int32),
            jax.ShapeDtypeStruct((_N, _D), jnp.float32),
        ],
    )(x, wn, we)


_NC = 2
_NS = 16
_NW = _NC * _NS
_KH = _K // 2
_CH = 120
_NCHUNK = 21
_NOUTER = _NCHUNK // 3
_BPW = _CH * _NCHUNK
_EPH = _BPW * _NW
_NPAD = _EPH // _KH


def _make_gather():
    mesh = plsc.VectorSubcoreMesh(core_axis_name="c", subcore_axis_name="s")

    @functools.partial(
        pl.kernel,
        mesh=mesh,
        out_type=[
            jax.ShapeDtypeStruct((_EPH, _NOU), jnp.int32),
            jax.ShapeDtypeStruct((_EPH, _NOU), jnp.int32),
        ],
        scratch_types=[
            pltpu.VMEM((_BPW,), jnp.int32),
            pltpu.VMEM((3, _CH, _NOU), jnp.int32),
            pltpu.VMEM((3, _CH, _NOU), jnp.int32),
            pltpu.SemaphoreType.DMA((3,)),
            pltpu.SemaphoreType.DMA((3,)),
        ],
    )
    def gather_k(ta_hbm, tb_hbm, idx_hbm, oa_hbm, ob_hbm, idx_v, ra_v, rb_v, gsem, wsem):
        wid = lax.axis_index("s") * _NC + lax.axis_index("c")
        base = wid * _BPW
        pltpu.sync_copy(idx_hbm.at[pl.ds(base, _BPW)], idx_v)

        def fire_gather(g, slot):
            sl = idx_v.at[pl.ds(g * _CH, _CH)]
            pltpu.async_copy(ta_hbm.at[sl], ra_v.at[slot], gsem.at[slot])
            pltpu.async_copy(tb_hbm.at[sl], rb_v.at[slot], gsem.at[slot])

        def wait_gather(g, slot):
            sl = idx_v.at[pl.ds(g * _CH, _CH)]
            pltpu.make_async_copy(ta_hbm.at[sl], ra_v.at[slot], gsem.at[slot]).wait()
            pltpu.make_async_copy(tb_hbm.at[sl], rb_v.at[slot], gsem.at[slot]).wait()

        def fire_write(g, slot):
            dst = pl.ds(base + g * _CH, _CH)
            pltpu.async_copy(ra_v.at[slot], oa_hbm.at[dst], wsem.at[slot])
            pltpu.async_copy(rb_v.at[slot], ob_hbm.at[dst], wsem.at[slot])

        def wait_write(slot):
            pltpu.make_async_copy(
                ra_v.at[slot], oa_hbm.at[pl.ds(0, _CH)], wsem.at[slot]
            ).wait()
            pltpu.make_async_copy(
                rb_v.at[slot], ob_hbm.at[pl.ds(0, _CH)], wsem.at[slot]
            ).wait()

        fire_gather(0, 0)
        fire_gather(1, 1)

        def outer(g3, carry):
            for b in range(3):
                g = g3 * 3 + b
                wait_gather(g, b)
                fire_write(g, b)
                nxt = (b + 2) % 3
                if b == 0:
                    @pl.when(g3 >= 1)
                    def _():
                        wait_write(nxt)
                    fire_gather(g + 2, nxt)
                else:
                    @pl.when(g3 < _NOUTER - 1)
                    def _():
                        wait_write(nxt)
                        fire_gather(g + 2, nxt)
            return carry

        lax.fori_loop(0, _NOUTER, outer, 0)
        for b in range(3):
            wait_write(b)

    return gather_k


_gather_cache = []


def _gather(table_a, table_b, idx):
    if not _gather_cache:
        _gather_cache.append(_make_gather())
    return _gather_cache[0](table_a, table_b, idx)


_NB_E = 200


def _edge_body(koff, s_ref, pa_ref, pb_ref, et_ref, bias_ref, out_ref):
    s = s_ref[...]
    et4 = et_ref[0]
    m_lo = m_hi = None
    for k in range(_KH):
        pka = pa_ref[k]
        pkb = pb_ref[k]
        t_lo = jnp.zeros((_NB_E, _DGE), jnp.float32)
        t_hi = jnp.zeros((_NB_E, _DGE), jnp.float32)
        for e in range(_ET):
            w = et4[e, :, koff + k:koff + k + 1]
            pk = pka if e < 2 else pkb
            word = pk[:, (e % 2) * _DGE:((e % 2) + 1) * _DGE]
            lo = lax.bitcast_convert_type(word << 16, jnp.float32)
            hi = lax.bitcast_convert_type(word & jnp.int32(-65536), jnp.float32)
            t_lo = t_lo + w * (s[:, e * _NOU:e * _NOU + _DGE] - lo)
            t_hi = t_hi + w * (s[:, e * _NOU + _DGE:(e + 1) * _NOU] - hi)
        if m_lo is None:
            m_lo, m_hi = t_lo, t_hi
        else:
            m_lo = jnp.maximum(m_lo, t_lo)
            m_hi = jnp.maximum(m_hi, t_hi)
    b = bias_ref[...]
    out_ref[:, :_DGE] = m_lo + b[:, :_DGE]
    out_ref[:, _DGE:] = m_hi + b[:, _DGE:]


def _edge(s, pts_a, pts_b, etype, bias2d, koff):
    return pl.pallas_call(
        functools.partial(_edge_body, koff),
        grid=(_N // _NB_E,),
        in_specs=[
            pl.BlockSpec((_NB_E, _D), lambda i: (i, 0)),
            pl.BlockSpec((_KH, _NB_E, _NOU), lambda i: (0, i, 0)),
            pl.BlockSpec((_KH, _NB_E, _NOU), lambda i: (0, i, 0)),
            pl.BlockSpec((1, _ET, _NB_E, _K), lambda i: (0, 0, i, 0)),
            pl.BlockSpec((1, _NOU), lambda i: (0, 0)),
        ],
        out_specs=pl.BlockSpec((_NB_E, _NOU), lambda i: (i, 0)),
        out_shape=jax.ShapeDtypeStruct((_N, _NOU), jnp.float32),
    )(s, pts_a, pts_b, etype, bias2d)


def _bn_body(o1_ref, o2_ref, g_ref, b_ref, out_ref):
    o = jnp.maximum(o1_ref[...], o2_ref[...])
    mean = jnp.mean(o, axis=0, keepdims=True)
    ctr = o - mean
    var = jnp.mean(ctr * ctr, axis=0, keepdims=True)
    y = ctr * lax.rsqrt(var + 1e-5) * g_ref[...] + b_ref[...]
    out_ref[...] = jnp.maximum(y, 0.0).T


def _bn(o1, o2, g, b):
    return pl.pallas_call(
        _bn_body,
        out_shape=jax.ShapeDtypeStruct((_NOU, _N), jnp.float32),
    )(o1, o2, g, b)


def kernel(x, nn_idx, etype, filters1, filters2, bias, bn_gamma, bn_beta):
    wn = jnp.transpose(filters1, (0, 2, 1)).reshape(_NIN, _D)[:, _PERM512]
    we = jnp.transpose(filters2, (0, 2, 1)).reshape(_NIN, _D)[:, _PERM512]
    ef_a, ef_b, s = _matmul(jnp.transpose(x.reshape(_NIN, _N)), wn, we)

    pad = (jnp.arange(_K * (_NPAD - _N), dtype=jnp.int32) * 7) % _N
    idx_km = jnp.concatenate(
        [
            jnp.transpose(nn_idx.reshape(_N, _K)).astype(jnp.int32),
            pad.reshape(_K, _NPAD - _N),
        ],
        axis=1,
    )
    bias_p = bias[_PERM].reshape(1, _NOU)

    p1a, p1b = _gather(ef_a, ef_b, idx_km[:_KH].reshape(-1))
    p2a, p2b = _gather(ef_a, ef_b, idx_km[_KH:].reshape(-1))
    sh = (_KH, _NPAD, _NOU)
    o1 = _edge(s, p1a.reshape(sh), p1b.reshape(sh), etype, bias_p, 0)
    o2 = _edge(s, p2a.reshape(sh), p2b.reshape(sh), etype, bias_p, _KH)
    out = _bn(
        o1,
        o2,
        bn_gamma[_PERM].reshape(1, _NOU),
        bn_beta[_PERM].reshape(1, _NOU),
    )
    return out[_INV_PERM, :].reshape(1, _NOU, _N, 1)
